# baseline (speedup 1.0000x reference)
import functools

import jax
import jax.numpy as jnp
from jax import lax
from jax.experimental import pallas as pl
from jax.experimental.pallas import tpu as pltpu
from jax.experimental.pallas import tpu_sc as plsc

BATCH = 8192
DM = 2048
DS = 16384
TOPK = 64

NEGINF = float("-inf")


_ENC_BM = 1024
_ENC_BN = 512


def _enc_body(x_ref, bpre_ref, w_ref, benc_ref, o_ref):
    xc = x_ref[...] - bpre_ref[...]
    acc = lax.dot_general(
        xc, w_ref[...], (((1,), (0,)), ((), ())),
        preferred_element_type=jnp.float32)
    o_ref[...] = acc + benc_ref[...]


def _encode(x, W_enc, b_enc, b_pre):
    grid = (BATCH // _ENC_BM, DS // _ENC_BN)
    return pl.pallas_call(
        _enc_body,
        grid=grid,
        in_specs=[
            pl.BlockSpec((_ENC_BM, DM), lambda i, j: (i, 0)),
            pl.BlockSpec((1, DM), lambda i, j: (0, 0)),
            pl.BlockSpec((DM, _ENC_BN), lambda i, j: (0, j)),
            pl.BlockSpec((1, _ENC_BN), lambda i, j: (0, j)),
        ],
        out_specs=pl.BlockSpec((_ENC_BM, _ENC_BN), lambda i, j: (i, j)),
        out_shape=jax.ShapeDtypeStruct((BATCH, DS), jnp.float32),
        compiler_params=pltpu.CompilerParams(
            dimension_semantics=("parallel", "parallel")),
    )(x, b_pre.reshape(1, DM), W_enc, b_enc.reshape(1, DS))


_DEC_BM = 512
_DEC_BK = 2048


def _dec_body(l_ref, w_ref, bpre_ref, o_ref):
    kk = pl.program_id(1)
    acc = lax.dot_general(
        l_ref[...].astype(jnp.bfloat16), w_ref[...],
        (((1,), (0,)), ((), ())),
        preferred_element_type=jnp.float32)

    @pl.when(kk == 0)
    def _():
        o_ref[...] = acc + bpre_ref[...]

    @pl.when(kk != 0)
    def _():
        o_ref[...] = o_ref[...] + acc


def _decode(dense, W_dec_bf16, b_pre):
    grid = (BATCH // _DEC_BM, DS // _DEC_BK)
    return pl.pallas_call(
        _dec_body,
        grid=grid,
        in_specs=[
            pl.BlockSpec((_DEC_BM, _DEC_BK), lambda i, kk: (i, kk)),
            pl.BlockSpec((_DEC_BK, DM), lambda i, kk: (kk, 0)),
            pl.BlockSpec((1, DM), lambda i, kk: (0, 0)),
        ],
        out_specs=pl.BlockSpec((_DEC_BM, DM), lambda i, kk: (i, 0)),
        out_shape=jax.ShapeDtypeStruct((BATCH, DM), jnp.float32),
        compiler_params=pltpu.CompilerParams(
            dimension_semantics=("parallel", "arbitrary")),
    )(dense, W_dec_bf16, b_pre.reshape(1, DM))


_NW = 32
_RPW = BATCH // _NW
_NSUP = 16


def _scal(v):
    return v[0] if getattr(v, "ndim", 0) else v


def _rmax(v):
    return plsc.cummax(v)[15]


def _sc_topk(latents):
    mesh = plsc.VectorSubcoreMesh(core_axis_name="c", subcore_axis_name="s")

    @functools.partial(
        pl.kernel,
        out_type=(
            jax.ShapeDtypeStruct((BATCH, TOPK), jnp.float32),
            jax.ShapeDtypeStruct((BATCH, TOPK), jnp.int32),
            jax.ShapeDtypeStruct((BATCH, DS), jnp.float32),
        ),
        mesh=mesh,
        scratch_types=[
            pltpu.VMEM((DS,), jnp.float32),
            pltpu.VMEM((DS,), jnp.float32),
            pltpu.VMEM((DS,), jnp.float32),
            pltpu.VMEM((DS,), jnp.float32),
            pltpu.VMEM((DS,), jnp.float32),
            pltpu.VMEM((_NSUP * 16,), jnp.float32),
            pltpu.VMEM((_NSUP * 16,), jnp.float32),
            pltpu.VMEM((2, TOPK), jnp.float32),
            pltpu.VMEM((2, TOPK), jnp.int32),
            pltpu.SemaphoreType.DMA,
            pltpu.SemaphoreType.DMA,
            pltpu.SemaphoreType.DMA,
            pltpu.SemaphoreType.DMA,
        ],
        compiler_params=pltpu.CompilerParams(needs_layout_passes=False),
    )
    def body(lat, vals, idx, dense, rb0, rb1, rb2, rb3, zbuf, lmaxA, lmaxB,
             vstage, istage, sem0, sem1, sem2, sem3):
        iota = lax.iota(jnp.int32, 16)
        zero16 = jnp.zeros((16,), jnp.float32)
        zero16i = jnp.zeros((16,), jnp.int32)
        neg16 = jnp.full((16,), NEGINF, jnp.float32)
        wid = lax.axis_index("s") * 2 + lax.axis_index("c")
        base = wid * _RPW

        def zero_body(i, carry):
            zbuf[pl.ds(i * 16, 16)] = zero16
            return carry

        lax.fori_loop(0, DS // 16, zero_body, 0)

        def pop_one(rbuf, lmax, sm, vv, ii, qi, qr):
            m = _rmax(sm)
            su = _scal(plsc.all_reduce_ffs(sm == m))
            lm = lmax[pl.ds(su * 16, 16)]
            lane = _scal(plsc.all_reduce_ffs(lm == m))
            cbase = su * 1024 + lane
            d = [plsc.load_gather(rbuf, [cbase + g * 256 + iota * 16])
                 for g in range(4)]
            jw = jnp.int32(1 << 20)
            for g in range(4):
                ff = _scal(plsc.all_reduce_ffs(d[g] == m))
                jw = jnp.minimum(jw, g * 16 + ff + ((ff & 16) << 8))
            gw = jw // 16
            tw = jw % 16
            flat = cbase + jw * 16
            for jj in range(4):
                sel = jnp.logical_and(qi == jj, iota == qr)
                vv[jj] = jnp.where(sel, m, vv[jj])
                ii[jj] = jnp.where(sel, flat, ii[jj])
            plsc.store_scatter(rbuf, [flat + iota * 0], neg16,
                               mask=iota == 0)
            cm = NEGINF
            for g in range(4):
                d2 = jnp.where(jnp.logical_and(gw == g, iota == tw),
                               NEGINF, d[g])
                cm = jnp.maximum(cm, _rmax(d2))
            lm2 = jnp.where(iota == lane, cm, lm)
            lmax[pl.ds(su * 16, 16)] = lm2
            sm = jnp.where(iota == su, _rmax(lm2), sm)
            return sm, vv, ii

        def do_pair(rbA, rbB, rowA, rowB, rlA, rlB):
            def p1(s, carry):
                smA, smB = carry
                off = s * 1024
                aA = [rbA[pl.ds(off + j * 16, 16)] for j in range(4)]
                aB = [rbB[pl.ds(off + j * 16, 16)] for j in range(4)]
                for j in range(4, 64):
                    aA[j % 4] = jnp.maximum(aA[j % 4],
                                            rbA[pl.ds(off + j * 16, 16)])
                    aB[j % 4] = jnp.maximum(aB[j % 4],
                                            rbB[pl.ds(off + j * 16, 16)])
                mvA = jnp.maximum(jnp.maximum(aA[0], aA[1]),
                                  jnp.maximum(aA[2], aA[3]))
                mvB = jnp.maximum(jnp.maximum(aB[0], aB[1]),
                                  jnp.maximum(aB[2], aB[3]))
                lmaxA[pl.ds(s * 16, 16)] = mvA
                lmaxB[pl.ds(s * 16, 16)] = mvB
                smA = jnp.where(iota == s, _rmax(mvA), smA)
                smB = jnp.where(iota == s, _rmax(mvB), smB)
                return smA, smB

            smA, smB = lax.fori_loop(0, _NSUP, p1, (neg16, neg16))

            def pop(i, carry):
                smA = carry[0]
                vvA = list(carry[1:5])
                iiA = list(carry[5:9])
                smB = carry[9]
                vvB = list(carry[10:14])
                iiB = list(carry[14:18])
                qi = i // 16
                qr = i % 16
                smA, vvA, iiA = pop_one(rbA, lmaxA, smA, vvA, iiA, qi, qr)
                smB, vvB, iiB = pop_one(rbB, lmaxB, smB, vvB, iiB, qi, qr)
                return (smA,) + tuple(vvA) + tuple(iiA) + \
                       (smB,) + tuple(vvB) + tuple(iiB)

            init = (smA, zero16, zero16, zero16, zero16,
                    zero16i, zero16i, zero16i, zero16i,
                    smB, zero16, zero16, zero16, zero16,
                    zero16i, zero16i, zero16i, zero16i)
            res = lax.fori_loop(0, TOPK, pop, init)

            for (sl, row, off) in ((0, rowA, 1), (1, rowB, 10)):
                for jj in range(TOPK // 16):
                    vstage[sl, pl.ds(jj * 16, 16)] = res[off + jj]
                    istage[sl, pl.ds(jj * 16, 16)] = res[off + 4 + jj]
                    plsc.store_scatter(zbuf, [res[off + 4 + jj]],
                                       res[off + jj])
                pltpu.sync_copy(zbuf, dense.at[row])
                for jj in range(TOPK // 16):
                    plsc.store_scatter(zbuf, [res[off + 4 + jj]], zero16)
            pltpu.sync_copy(vstage, vals.at[pl.ds(rowA, 2)])
            pltpu.sync_copy(istage, idx.at[pl.ds(rowA, 2)])

        pltpu.async_copy(lat.at[base], rb0, sem0)
        pltpu.async_copy(lat.at[base + 1], rb1, sem1)

        def outer(o, carry):
            r = o * 4
            pltpu.async_copy(lat.at[base + r + 2], rb2, sem2)
            pltpu.async_copy(lat.at[base + r + 3], rb3, sem3)
            pltpu.make_async_copy(lat.at[base + r], rb0, sem0).wait()
            pltpu.make_async_copy(lat.at[base + r + 1], rb1, sem1).wait()
            do_pair(rb0, rb1, base + r, base + r + 1, r, r + 1)

            @pl.when(o < _RPW // 4 - 1)
            def _():
                pltpu.async_copy(lat.at[base + r + 4], rb0, sem0)
                pltpu.async_copy(lat.at[base + r + 5], rb1, sem1)

            pltpu.make_async_copy(lat.at[base + r + 2], rb2, sem2).wait()
            pltpu.make_async_copy(lat.at[base + r + 3], rb3, sem3).wait()
            do_pair(rb2, rb3, base + r + 2, base + r + 3, r + 2, r + 3)
            return carry

        lax.fori_loop(0, _RPW // 4, outer, 0)

    return body(latents)


def kernel(x, W_enc, b_enc, W_dec, b_pre, k):
    del k
    latents = _encode(x, W_enc, b_enc, b_pre)
    vals, idxs, dense = _sc_topk(latents)
    x_hat = _decode(dense, W_dec.astype(jnp.bfloat16), b_pre)
    return x_hat, dense, idxs, vals


# speedup vs baseline: 1.1537x; 1.1537x over previous
import functools

import jax
import jax.numpy as jnp
from jax import lax
from jax.experimental import pallas as pl
from jax.experimental.pallas import tpu as pltpu
from jax.experimental.pallas import tpu_sc as plsc

BATCH = 8192
DM = 2048
DS = 16384
TOPK = 64

NEGINF = float("-inf")


_ENC_BM = 1024
_ENC_BN = 512


def _enc_body(x_ref, bpre_ref, w_ref, benc_ref, o_ref):
    xc = x_ref[...] - bpre_ref[...]
    acc = lax.dot_general(
        xc, w_ref[...], (((1,), (0,)), ((), ())),
        preferred_element_type=jnp.float32)
    o_ref[...] = acc + benc_ref[...]


def _encode(x, W_enc, b_enc, b_pre, rows):
    grid = (rows // _ENC_BM, DS // _ENC_BN)
    return pl.pallas_call(
        _enc_body,
        grid=grid,
        in_specs=[
            pl.BlockSpec((_ENC_BM, DM), lambda i, j: (i, 0)),
            pl.BlockSpec((1, DM), lambda i, j: (0, 0)),
            pl.BlockSpec((DM, _ENC_BN), lambda i, j: (0, j)),
            pl.BlockSpec((1, _ENC_BN), lambda i, j: (0, j)),
        ],
        out_specs=pl.BlockSpec((_ENC_BM, _ENC_BN), lambda i, j: (i, j)),
        out_shape=jax.ShapeDtypeStruct((rows, DS), jnp.float32),
        compiler_params=pltpu.CompilerParams(
            dimension_semantics=("parallel", "parallel")),
    )(x, b_pre.reshape(1, DM), W_enc, b_enc.reshape(1, DS))


_DEC_BM = 512
_DEC_BK = 2048


def _dec_body_first(l_ref, w_ref, bpre_ref, xh_ref, dfull_ref):
    kk = pl.program_id(1)
    blk = l_ref[...]
    dfull_ref[...] = blk
    acc = lax.dot_general(
        blk.astype(jnp.bfloat16), w_ref[...],
        (((1,), (0,)), ((), ())),
        preferred_element_type=jnp.float32)

    @pl.when(kk == 0)
    def _():
        xh_ref[...] = acc + bpre_ref[...]

    @pl.when(kk != 0)
    def _():
        xh_ref[...] = xh_ref[...] + acc


def _dec_body_chain(l_ref, w_ref, bpre_ref, dprev_ref, xh_ref, dfull_ref):
    del dprev_ref
    _dec_body_first(l_ref, w_ref, bpre_ref, xh_ref, dfull_ref)


def _decode_slice(dense_s, W_dec_bf16, b_pre, dense_prev, s, rows):
    grid = (rows // _DEC_BM, DS // _DEC_BK)
    row0 = s * rows // _DEC_BM
    in_specs = [
        pl.BlockSpec((_DEC_BM, _DEC_BK), lambda i, kk: (i, kk)),
        pl.BlockSpec((_DEC_BK, DM), lambda i, kk: (kk, 0)),
        pl.BlockSpec((1, DM), lambda i, kk: (0, 0)),
    ]
    out_specs = (
        pl.BlockSpec((_DEC_BM, DM), lambda i, kk: (i, 0)),
        pl.BlockSpec((_DEC_BM, _DEC_BK), lambda i, kk: (row0 + i, kk)),
    )
    out_shape = (
        jax.ShapeDtypeStruct((rows, DM), jnp.float32),
        jax.ShapeDtypeStruct((BATCH, DS), jnp.float32),
    )
    cp = pltpu.CompilerParams(dimension_semantics=("parallel", "arbitrary"))
    args = [dense_s, W_dec_bf16, b_pre.reshape(1, DM)]
    if dense_prev is None:
        return pl.pallas_call(
            _dec_body_first, grid=grid, in_specs=in_specs,
            out_specs=out_specs, out_shape=out_shape, compiler_params=cp,
        )(*args)
    return pl.pallas_call(
        _dec_body_chain, grid=grid,
        in_specs=in_specs + [pl.BlockSpec(memory_space=pl.ANY)],
        out_specs=out_specs, out_shape=out_shape,
        input_output_aliases={3: 1}, compiler_params=cp,
    )(*args, dense_prev)


_NW = 32
_RPW = BATCH // _NW
_NSUP = 16


def _scal(v):
    return v[0] if getattr(v, "ndim", 0) else v


def _rmax(v):
    return plsc.cummax(v)[15]


def _sc_topk(latents, rows):
    rpw = rows // _NW
    mesh = plsc.VectorSubcoreMesh(core_axis_name="c", subcore_axis_name="s")

    @functools.partial(
        pl.kernel,
        out_type=(
            jax.ShapeDtypeStruct((rows, TOPK), jnp.float32),
            jax.ShapeDtypeStruct((rows, TOPK), jnp.int32),
            jax.ShapeDtypeStruct((rows, DS), jnp.float32),
        ),
        mesh=mesh,
        scratch_types=[
            pltpu.VMEM((DS,), jnp.float32),
            pltpu.VMEM((DS,), jnp.float32),
            pltpu.VMEM((DS,), jnp.float32),
            pltpu.VMEM((DS,), jnp.float32),
            pltpu.VMEM((DS,), jnp.float32),
            pltpu.VMEM((_NSUP * 16,), jnp.float32),
            pltpu.VMEM((_NSUP * 16,), jnp.float32),
            pltpu.VMEM((2, TOPK), jnp.float32),
            pltpu.VMEM((2, TOPK), jnp.int32),
            pltpu.SemaphoreType.DMA,
            pltpu.SemaphoreType.DMA,
            pltpu.SemaphoreType.DMA,
            pltpu.SemaphoreType.DMA,
        ],
        compiler_params=pltpu.CompilerParams(needs_layout_passes=False),
    )
    def body(lat, vals, idx, dense, rb0, rb1, rb2, rb3, zbuf, lmaxA, lmaxB,
             vstage, istage, sem0, sem1, sem2, sem3):
        iota = lax.iota(jnp.int32, 16)
        zero16 = jnp.zeros((16,), jnp.float32)
        zero16i = jnp.zeros((16,), jnp.int32)
        neg16 = jnp.full((16,), NEGINF, jnp.float32)
        wid = lax.axis_index("s") * 2 + lax.axis_index("c")
        base = wid * rpw

        def zero_body(i, carry):
            zbuf[pl.ds(i * 16, 16)] = zero16
            return carry

        lax.fori_loop(0, DS // 16, zero_body, 0)

        def pop_one(rbuf, lmax, sm, vv, ii, qi, qr):
            m = _rmax(sm)
            su = _scal(plsc.all_reduce_ffs(sm == m))
            lm = lmax[pl.ds(su * 16, 16)]
            lane = _scal(plsc.all_reduce_ffs(lm == m))
            cbase = su * 1024 + lane
            d = [plsc.load_gather(rbuf, [cbase + g * 256 + iota * 16])
                 for g in range(4)]
            jw = jnp.int32(1 << 20)
            for g in range(4):
                ff = _scal(plsc.all_reduce_ffs(d[g] == m))
                jw = jnp.minimum(jw, g * 16 + ff + ((ff & 16) << 8))
            gw = jw // 16
            tw = jw % 16
            flat = cbase + jw * 16
            for jj in range(4):
                sel = jnp.logical_and(qi == jj, iota == qr)
                vv[jj] = jnp.where(sel, m, vv[jj])
                ii[jj] = jnp.where(sel, flat, ii[jj])
            plsc.store_scatter(rbuf, [flat + iota * 0], neg16,
                               mask=iota == 0)
            d2 = [jnp.where(jnp.logical_and(gw == g, iota == tw),
                            NEGINF, d[g]) for g in range(4)]
            cm = _rmax(jnp.maximum(jnp.maximum(d2[0], d2[1]),
                                   jnp.maximum(d2[2], d2[3])))
            lm2 = jnp.where(iota == lane, cm, lm)
            lmax[pl.ds(su * 16, 16)] = lm2
            sm = jnp.where(iota == su, _rmax(lm2), sm)
            return sm, vv, ii

        def do_pair(rbA, rbB, rowA, rowB, rlA, rlB):
            def p1(s, carry):
                smA, smB = carry
                off = s * 1024
                aA = [rbA[pl.ds(off + j * 16, 16)] for j in range(4)]
                aB = [rbB[pl.ds(off + j * 16, 16)] for j in range(4)]
                for j in range(4, 64):
                    aA[j % 4] = jnp.maximum(aA[j % 4],
                                            rbA[pl.ds(off + j * 16, 16)])
                    aB[j % 4] = jnp.maximum(aB[j % 4],
                                            rbB[pl.ds(off + j * 16, 16)])
                mvA = jnp.maximum(jnp.maximum(aA[0], aA[1]),
                                  jnp.maximum(aA[2], aA[3]))
                mvB = jnp.maximum(jnp.maximum(aB[0], aB[1]),
                                  jnp.maximum(aB[2], aB[3]))
                lmaxA[pl.ds(s * 16, 16)] = mvA
                lmaxB[pl.ds(s * 16, 16)] = mvB
                smA = jnp.where(iota == s, _rmax(mvA), smA)
                smB = jnp.where(iota == s, _rmax(mvB), smB)
                return smA, smB

            smA, smB = lax.fori_loop(0, _NSUP, p1, (neg16, neg16))

            def pop(i, carry):
                smA = carry[0]
                vvA = list(carry[1:5])
                iiA = list(carry[5:9])
                smB = carry[9]
                vvB = list(carry[10:14])
                iiB = list(carry[14:18])
                qi = i // 16
                qr = i % 16
                smA, vvA, iiA = pop_one(rbA, lmaxA, smA, vvA, iiA, qi, qr)
                smB, vvB, iiB = pop_one(rbB, lmaxB, smB, vvB, iiB, qi, qr)
                return (smA,) + tuple(vvA) + tuple(iiA) + \
                       (smB,) + tuple(vvB) + tuple(iiB)

            init = (smA, zero16, zero16, zero16, zero16,
                    zero16i, zero16i, zero16i, zero16i,
                    smB, zero16, zero16, zero16, zero16,
                    zero16i, zero16i, zero16i, zero16i)
            res = lax.fori_loop(0, TOPK, pop, init)

            for (sl, row, off) in ((0, rowA, 1), (1, rowB, 10)):
                for jj in range(TOPK // 16):
                    vstage[sl, pl.ds(jj * 16, 16)] = res[off + jj]
                    istage[sl, pl.ds(jj * 16, 16)] = res[off + 4 + jj]
                    plsc.store_scatter(zbuf, [res[off + 4 + jj]],
                                       res[off + jj])
                pltpu.sync_copy(zbuf, dense.at[row])
                for jj in range(TOPK // 16):
                    plsc.store_scatter(zbuf, [res[off + 4 + jj]], zero16)
            pltpu.sync_copy(vstage, vals.at[pl.ds(rowA, 2)])
            pltpu.sync_copy(istage, idx.at[pl.ds(rowA, 2)])

        pltpu.async_copy(lat.at[base], rb0, sem0)
        pltpu.async_copy(lat.at[base + 1], rb1, sem1)

        def outer(o, carry):
            r = o * 4
            pltpu.async_copy(lat.at[base + r + 2], rb2, sem2)
            pltpu.async_copy(lat.at[base + r + 3], rb3, sem3)
            pltpu.make_async_copy(lat.at[base + r], rb0, sem0).wait()
            pltpu.make_async_copy(lat.at[base + r + 1], rb1, sem1).wait()
            do_pair(rb0, rb1, base + r, base + r + 1, r, r + 1)

            @pl.when(o < rpw // 4 - 1)
            def _():
                pltpu.async_copy(lat.at[base + r + 4], rb0, sem0)
                pltpu.async_copy(lat.at[base + r + 5], rb1, sem1)

            pltpu.make_async_copy(lat.at[base + r + 2], rb2, sem2).wait()
            pltpu.make_async_copy(lat.at[base + r + 3], rb3, sem3).wait()
            do_pair(rb2, rb3, base + r + 2, base + r + 3, r + 2, r + 3)
            return carry

        lax.fori_loop(0, rpw // 4, outer, 0)

    return body(latents)


_NSLICE = 4


def kernel(x, W_enc, b_enc, W_dec, b_pre, k):
    del k
    rs = BATCH // _NSLICE
    Wd = W_dec.astype(jnp.bfloat16)
    vals_l, idx_l, xh_l = [], [], []
    dense_full = None
    for s in range(_NSLICE):
        lat = _encode(x[s * rs:(s + 1) * rs], W_enc, b_enc, b_pre, rs)
        v, i, d = _sc_topk(lat, rs)
        xh, dense_full = _decode_slice(d, Wd, b_pre, dense_full, s, rs)
        vals_l.append(v)
        idx_l.append(i)
        xh_l.append(xh)
    return (jnp.concatenate(xh_l), dense_full,
            jnp.concatenate(idx_l), jnp.concatenate(vals_l))


# speedup vs baseline: 1.2269x; 1.0635x over previous
import functools

import jax
import jax.numpy as jnp
from jax import lax
from jax.experimental import pallas as pl
from jax.experimental.pallas import tpu as pltpu
from jax.experimental.pallas import tpu_sc as plsc

BATCH = 8192
DM = 2048
DS = 16384
TOPK = 64

NEGINF = float("-inf")


_ENC_BM = 1024
_ENC_BN = 512


def _enc_body(x_ref, bpre_ref, w_ref, benc_ref, o_ref):
    xc = x_ref[...] - bpre_ref[...]
    acc = lax.dot_general(
        xc, w_ref[...], (((1,), (0,)), ((), ())),
        preferred_element_type=jnp.float32)
    o_ref[...] = acc + benc_ref[...]


def _encode(x, W_enc, b_enc, b_pre, rows):
    grid = (rows // _ENC_BM, DS // _ENC_BN)
    return pl.pallas_call(
        _enc_body,
        grid=grid,
        in_specs=[
            pl.BlockSpec((_ENC_BM, DM), lambda i, j: (i, 0)),
            pl.BlockSpec((1, DM), lambda i, j: (0, 0)),
            pl.BlockSpec((DM, _ENC_BN), lambda i, j: (0, j)),
            pl.BlockSpec((1, _ENC_BN), lambda i, j: (0, j)),
        ],
        out_specs=pl.BlockSpec((_ENC_BM, _ENC_BN), lambda i, j: (i, j)),
        out_shape=jax.ShapeDtypeStruct((rows, DS), jnp.float32),
        compiler_params=pltpu.CompilerParams(
            dimension_semantics=("parallel", "parallel")),
    )(x, b_pre.reshape(1, DM), W_enc, b_enc.reshape(1, DS))


_DEC_BM = 512
_DEC_BK = 2048


def _dec_body_first(l_ref, w_ref, bpre_ref, xh_ref, dfull_ref):
    kk = pl.program_id(1)
    blk = l_ref[...]
    dfull_ref[...] = blk
    acc = lax.dot_general(
        blk.astype(jnp.bfloat16), w_ref[...],
        (((1,), (0,)), ((), ())),
        preferred_element_type=jnp.float32)

    @pl.when(kk == 0)
    def _():
        xh_ref[...] = acc + bpre_ref[...]

    @pl.when(kk != 0)
    def _():
        xh_ref[...] = xh_ref[...] + acc


def _dec_body_chain(l_ref, w_ref, bpre_ref, dprev_ref, xh_ref, dfull_ref):
    del dprev_ref
    _dec_body_first(l_ref, w_ref, bpre_ref, xh_ref, dfull_ref)


def _decode_slice(dense_s, W_dec_bf16, b_pre, dense_prev, s, rows):
    grid = (rows // _DEC_BM, DS // _DEC_BK)
    row0 = s * rows // _DEC_BM
    in_specs = [
        pl.BlockSpec((_DEC_BM, _DEC_BK), lambda i, kk: (i, kk)),
        pl.BlockSpec((_DEC_BK, DM), lambda i, kk: (kk, 0)),
        pl.BlockSpec((1, DM), lambda i, kk: (0, 0)),
    ]
    out_specs = (
        pl.BlockSpec((_DEC_BM, DM), lambda i, kk: (i, 0)),
        pl.BlockSpec((_DEC_BM, _DEC_BK), lambda i, kk: (row0 + i, kk)),
    )
    out_shape = (
        jax.ShapeDtypeStruct((rows, DM), jnp.float32),
        jax.ShapeDtypeStruct((BATCH, DS), jnp.float32),
    )
    cp = pltpu.CompilerParams(dimension_semantics=("parallel", "arbitrary"))
    args = [dense_s, W_dec_bf16, b_pre.reshape(1, DM)]
    if dense_prev is None:
        return pl.pallas_call(
            _dec_body_first, grid=grid, in_specs=in_specs,
            out_specs=out_specs, out_shape=out_shape, compiler_params=cp,
        )(*args)
    return pl.pallas_call(
        _dec_body_chain, grid=grid,
        in_specs=in_specs + [pl.BlockSpec(memory_space=pl.ANY)],
        out_specs=out_specs, out_shape=out_shape,
        input_output_aliases={3: 1}, compiler_params=cp,
    )(*args, dense_prev)


_NW = 32
_RPW = BATCH // _NW
_NSUP = 16


def _scal(v):
    return v[0] if getattr(v, "ndim", 0) else v


def _rmax(v):
    return plsc.cummax(v)[15]


def _sc_topk(latents, rows):
    rpw = rows // _NW
    mesh = plsc.VectorSubcoreMesh(core_axis_name="c", subcore_axis_name="s")

    @functools.partial(
        pl.kernel,
        out_type=(
            jax.ShapeDtypeStruct((rows, TOPK), jnp.float32),
            jax.ShapeDtypeStruct((rows, TOPK), jnp.int32),
            jax.ShapeDtypeStruct((rows, DS), jnp.float32),
        ),
        mesh=mesh,
        scratch_types=[
            pltpu.VMEM((DS,), jnp.float32),
            pltpu.VMEM((DS,), jnp.float32),
            pltpu.VMEM((DS,), jnp.float32),
            pltpu.VMEM((DS,), jnp.float32),
            pltpu.VMEM((DS,), jnp.float32),
            pltpu.VMEM((DS,), jnp.float32),
            pltpu.VMEM((_NSUP * 16,), jnp.float32),
            pltpu.VMEM((_NSUP * 16,), jnp.float32),
            pltpu.VMEM((2, TOPK), jnp.float32),
            pltpu.VMEM((2, TOPK), jnp.int32),
            pltpu.VMEM((2, TOPK), jnp.float32),
            pltpu.VMEM((2, TOPK), jnp.int32),
            pltpu.SemaphoreType.DMA,
            pltpu.SemaphoreType.DMA,
            pltpu.SemaphoreType.DMA,
            pltpu.SemaphoreType.DMA,
            pltpu.SemaphoreType.DMA,
            pltpu.SemaphoreType.DMA,
            pltpu.SemaphoreType.DMA,
            pltpu.SemaphoreType.DMA,
        ],
        compiler_params=pltpu.CompilerParams(needs_layout_passes=False),
    )
    def body(lat, vals, idx, dense, rb0, rb1, rb2, rb3, zbuf0, zbuf1,
             lmaxA, lmaxB, svb0, sib0, svb1, sib1,
             sem0, sem1, sem2, sem3, semD0, semD1, semS0, semS1):
        iota = lax.iota(jnp.int32, 16)
        zero16 = jnp.zeros((16,), jnp.float32)
        zero16i = jnp.zeros((16,), jnp.int32)
        neg16 = jnp.full((16,), NEGINF, jnp.float32)
        wid = lax.axis_index("s") * 2 + lax.axis_index("c")
        base = wid * rpw

        def zero_body(i, carry):
            zbuf0[pl.ds(i * 16, 16)] = zero16
            zbuf1[pl.ds(i * 16, 16)] = zero16
            return carry

        lax.fori_loop(0, DS // 16, zero_body, 0)
        for jj in range(TOPK // 16):
            sib0[0, pl.ds(jj * 16, 16)] = zero16i
            sib0[1, pl.ds(jj * 16, 16)] = zero16i
            sib1[0, pl.ds(jj * 16, 16)] = zero16i
            sib1[1, pl.ds(jj * 16, 16)] = zero16i

        def pop_one(rbuf, lmax, sm, vv, ii, qi, qr):
            m = _rmax(sm)
            su = _scal(plsc.all_reduce_ffs(sm == m))
            lm = lmax[pl.ds(su * 16, 16)]
            lane = _scal(plsc.all_reduce_ffs(lm == m))
            cbase = su * 1024 + lane
            d = [plsc.load_gather(rbuf, [cbase + g * 256 + iota * 16])
                 for g in range(4)]
            jw = jnp.int32(1 << 20)
            for g in range(4):
                ff = _scal(plsc.all_reduce_ffs(d[g] == m))
                jw = jnp.minimum(jw, g * 16 + ff + ((ff & 16) << 8))
            gw = jw // 16
            tw = jw % 16
            flat = cbase + jw * 16
            for jj in range(4):
                sel = jnp.logical_and(qi == jj, iota == qr)
                vv[jj] = jnp.where(sel, m, vv[jj])
                ii[jj] = jnp.where(sel, flat, ii[jj])
            plsc.store_scatter(rbuf, [flat + iota * 0], neg16,
                               mask=iota == 0)
            d2 = [jnp.where(jnp.logical_and(gw == g, iota == tw),
                            NEGINF, d[g]) for g in range(4)]
            cm = _rmax(jnp.maximum(jnp.maximum(d2[0], d2[1]),
                                   jnp.maximum(d2[2], d2[3])))
            lm2 = jnp.where(iota == lane, cm, lm)
            lmax[pl.ds(su * 16, 16)] = lm2
            sm = jnp.where(iota == su, _rmax(lm2), sm)
            return sm, vv, ii

        def do_pair(rbA, rbB, rowA, rowB, sv, si, semS, sip,
                    dense_guard, stage_guard):
            def p1(s, carry):
                smA, smB = carry
                off = s * 1024
                aA = [rbA[pl.ds(off + j * 16, 16)] for j in range(4)]
                aB = [rbB[pl.ds(off + j * 16, 16)] for j in range(4)]
                for j in range(4, 64):
                    aA[j % 4] = jnp.maximum(aA[j % 4],
                                            rbA[pl.ds(off + j * 16, 16)])
                    aB[j % 4] = jnp.maximum(aB[j % 4],
                                            rbB[pl.ds(off + j * 16, 16)])
                mvA = jnp.maximum(jnp.maximum(aA[0], aA[1]),
                                  jnp.maximum(aA[2], aA[3]))
                mvB = jnp.maximum(jnp.maximum(aB[0], aB[1]),
                                  jnp.maximum(aB[2], aB[3]))
                lmaxA[pl.ds(s * 16, 16)] = mvA
                lmaxB[pl.ds(s * 16, 16)] = mvB
                smA = jnp.where(iota == s, _rmax(mvA), smA)
                smB = jnp.where(iota == s, _rmax(mvB), smB)
                return smA, smB

            smA, smB = lax.fori_loop(0, _NSUP, p1, (neg16, neg16))

            def pop(i, carry):
                smA = carry[0]
                vvA = list(carry[1:5])
                iiA = list(carry[5:9])
                smB = carry[9]
                vvB = list(carry[10:14])
                iiB = list(carry[14:18])
                qi = i // 16
                qr = i % 16
                smA, vvA, iiA = pop_one(rbA, lmaxA, smA, vvA, iiA, qi, qr)
                smB, vvB, iiB = pop_one(rbB, lmaxB, smB, vvB, iiB, qi, qr)
                return (smA,) + tuple(vvA) + tuple(iiA) + \
                       (smB,) + tuple(vvB) + tuple(iiB)

            init = (smA, zero16, zero16, zero16, zero16,
                    zero16i, zero16i, zero16i, zero16i,
                    smB, zero16, zero16, zero16, zero16,
                    zero16i, zero16i, zero16i, zero16i)
            res = lax.fori_loop(0, TOPK, pop, init)

            def wait_dense():
                pltpu.make_async_copy(zbuf0, dense.at[rowA], semD0).wait()
                pltpu.make_async_copy(zbuf1, dense.at[rowB], semD1).wait()

            if dense_guard is None:
                wait_dense()
            else:
                pl.when(dense_guard)(wait_dense)

            @pl.when(stage_guard)
            def _():
                pltpu.make_async_copy(sv, vals.at[pl.ds(rowA, 2)],
                                      semS).wait()
                pltpu.make_async_copy(si, idx.at[pl.ds(rowA, 2)],
                                      semS).wait()

            for jj in range(TOPK // 16):
                plsc.store_scatter(zbuf0, [sip[0, pl.ds(jj * 16, 16)]],
                                   zero16)
                plsc.store_scatter(zbuf1, [sip[1, pl.ds(jj * 16, 16)]],
                                   zero16)

            for (sl, zb, off) in ((0, zbuf0, 1), (1, zbuf1, 10)):
                for jj in range(TOPK // 16):
                    sv[sl, pl.ds(jj * 16, 16)] = res[off + jj]
                    si[sl, pl.ds(jj * 16, 16)] = res[off + 4 + jj]
                    plsc.store_scatter(zb, [res[off + 4 + jj]],
                                       res[off + jj])
            pltpu.async_copy(zbuf0, dense.at[rowA], semD0)
            pltpu.async_copy(zbuf1, dense.at[rowB], semD1)
            pltpu.async_copy(sv, vals.at[pl.ds(rowA, 2)], semS)
            pltpu.async_copy(si, idx.at[pl.ds(rowA, 2)], semS)

        pltpu.async_copy(lat.at[base], rb0, sem0)
        pltpu.async_copy(lat.at[base + 1], rb1, sem1)

        def outer(o, carry):
            r = o * 4
            pltpu.async_copy(lat.at[base + r + 2], rb2, sem2)
            pltpu.async_copy(lat.at[base + r + 3], rb3, sem3)
            pltpu.make_async_copy(lat.at[base + r], rb0, sem0).wait()
            pltpu.make_async_copy(lat.at[base + r + 1], rb1, sem1).wait()
            do_pair(rb0, rb1, base + r, base + r + 1,
                    svb0, sib0, semS0, sib1, o > 0, o > 0)

            @pl.when(o < rpw // 4 - 1)
            def _():
                pltpu.async_copy(lat.at[base + r + 4], rb0, sem0)
                pltpu.async_copy(lat.at[base + r + 5], rb1, sem1)

            pltpu.make_async_copy(lat.at[base + r + 2], rb2, sem2).wait()
            pltpu.make_async_copy(lat.at[base + r + 3], rb3, sem3).wait()
            do_pair(rb2, rb3, base + r + 2, base + r + 3,
                    svb1, sib1, semS1, sib0, None, o > 0)
            return carry

        lax.fori_loop(0, rpw // 4, outer, 0)

        pltpu.make_async_copy(zbuf0, dense.at[base], semD0).wait()
        pltpu.make_async_copy(zbuf1, dense.at[base], semD1).wait()
        pltpu.make_async_copy(svb0, vals.at[pl.ds(base, 2)], semS0).wait()
        pltpu.make_async_copy(sib0, idx.at[pl.ds(base, 2)], semS0).wait()
        pltpu.make_async_copy(svb1, vals.at[pl.ds(base, 2)], semS1).wait()
        pltpu.make_async_copy(sib1, idx.at[pl.ds(base, 2)], semS1).wait()

    return body(latents)


_NSLICE = 4


def kernel(x, W_enc, b_enc, W_dec, b_pre, k):
    del k
    rs = BATCH // _NSLICE
    Wd = W_dec.astype(jnp.bfloat16)
    vals_l, idx_l, xh_l = [], [], []
    dense_full = None
    for s in range(_NSLICE):
        lat = _encode(x[s * rs:(s + 1) * rs], W_enc, b_enc, b_pre, rs)
        v, i, d = _sc_topk(lat, rs)
        xh, dense_full = _decode_slice(d, Wd, b_pre, dense_full, s, rs)
        vals_l.append(v)
        idx_l.append(i)
        xh_l.append(xh)
    return (jnp.concatenate(xh_l), dense_full,
            jnp.concatenate(idx_l), jnp.concatenate(vals_l))


# speedup vs baseline: 1.2685x; 1.0339x over previous
import functools

import jax
import jax.numpy as jnp
from jax import lax
from jax.experimental import pallas as pl
from jax.experimental.pallas import tpu as pltpu
from jax.experimental.pallas import tpu_sc as plsc

BATCH = 8192
DM = 2048
DS = 16384
TOPK = 64

NEGINF = float("-inf")


_ENC_BM = 1024
_ENC_BN = 512


def _enc_body(x_ref, bpre_ref, w_ref, benc_ref, o_ref):
    xc = x_ref[...] - bpre_ref[...]
    acc = lax.dot_general(
        xc, w_ref[...], (((1,), (0,)), ((), ())),
        preferred_element_type=jnp.float32)
    o_ref[...] = acc + benc_ref[...]


def _encode(x, W_enc, b_enc, b_pre, rows):
    grid = (rows // _ENC_BM, DS // _ENC_BN)
    return pl.pallas_call(
        _enc_body,
        grid=grid,
        in_specs=[
            pl.BlockSpec((_ENC_BM, DM), lambda i, j: (i, 0)),
            pl.BlockSpec((1, DM), lambda i, j: (0, 0)),
            pl.BlockSpec((DM, _ENC_BN), lambda i, j: (0, j)),
            pl.BlockSpec((1, _ENC_BN), lambda i, j: (0, j)),
        ],
        out_specs=pl.BlockSpec((_ENC_BM, _ENC_BN), lambda i, j: (i, j)),
        out_shape=jax.ShapeDtypeStruct((rows, DS), jnp.float32),
        compiler_params=pltpu.CompilerParams(
            dimension_semantics=("parallel", "parallel")),
    )(x, b_pre.reshape(1, DM), W_enc, b_enc.reshape(1, DS))


_DEC_BM = 512
_DEC_BK = 2048


def _dec_body_first(l_ref, w_ref, bpre_ref, xh_ref, dfull_ref):
    kk = pl.program_id(1)
    blk = l_ref[...]
    dfull_ref[...] = blk
    acc = lax.dot_general(
        blk.astype(jnp.bfloat16), w_ref[...],
        (((1,), (0,)), ((), ())),
        preferred_element_type=jnp.float32)

    @pl.when(kk == 0)
    def _():
        xh_ref[...] = acc + bpre_ref[...]

    @pl.when(kk != 0)
    def _():
        xh_ref[...] = xh_ref[...] + acc


def _dec_body_chain(l_ref, w_ref, bpre_ref, dprev_ref, xh_ref, dfull_ref):
    del dprev_ref
    _dec_body_first(l_ref, w_ref, bpre_ref, xh_ref, dfull_ref)


def _decode_slice(dense_s, W_dec_bf16, b_pre, dense_prev, s, rows):
    grid = (rows // _DEC_BM, DS // _DEC_BK)
    row0 = s * rows // _DEC_BM
    in_specs = [
        pl.BlockSpec((_DEC_BM, _DEC_BK), lambda i, kk: (i, kk)),
        pl.BlockSpec((_DEC_BK, DM), lambda i, kk: (kk, 0)),
        pl.BlockSpec((1, DM), lambda i, kk: (0, 0)),
    ]
    out_specs = (
        pl.BlockSpec((_DEC_BM, DM), lambda i, kk: (i, 0)),
        pl.BlockSpec((_DEC_BM, _DEC_BK), lambda i, kk: (row0 + i, kk)),
    )
    out_shape = (
        jax.ShapeDtypeStruct((rows, DM), jnp.float32),
        jax.ShapeDtypeStruct((BATCH, DS), jnp.float32),
    )
    cp = pltpu.CompilerParams(dimension_semantics=("parallel", "arbitrary"))
    args = [dense_s, W_dec_bf16, b_pre.reshape(1, DM)]
    if dense_prev is None:
        return pl.pallas_call(
            _dec_body_first, grid=grid, in_specs=in_specs,
            out_specs=out_specs, out_shape=out_shape, compiler_params=cp,
        )(*args)
    return pl.pallas_call(
        _dec_body_chain, grid=grid,
        in_specs=in_specs + [pl.BlockSpec(memory_space=pl.ANY)],
        out_specs=out_specs, out_shape=out_shape,
        input_output_aliases={3: 1}, compiler_params=cp,
    )(*args, dense_prev)


_NW = 32
_RPW = BATCH // _NW
_NSUP = 16


def _scal(v):
    return v[0] if getattr(v, "ndim", 0) else v


def _rmax(v):
    return plsc.cummax(v)[15]


def _sc_topk(latents, rows):
    rpw = rows // _NW
    mesh = plsc.VectorSubcoreMesh(core_axis_name="c", subcore_axis_name="s")

    @functools.partial(
        pl.kernel,
        out_type=(
            jax.ShapeDtypeStruct((rows, TOPK), jnp.float32),
            jax.ShapeDtypeStruct((rows, TOPK), jnp.int32),
            jax.ShapeDtypeStruct((rows, DS), jnp.float32),
        ),
        mesh=mesh,
        scratch_types=[
            pltpu.VMEM((DS,), jnp.float32),
            pltpu.VMEM((DS,), jnp.float32),
            pltpu.VMEM((DS,), jnp.float32),
            pltpu.VMEM((DS,), jnp.float32),
            pltpu.VMEM((DS,), jnp.float32),
            pltpu.VMEM((DS,), jnp.float32),
            pltpu.VMEM((_NSUP * 16,), jnp.float32),
            pltpu.VMEM((_NSUP * 16,), jnp.float32),
            pltpu.VMEM((2, TOPK), jnp.float32),
            pltpu.VMEM((2, TOPK), jnp.int32),
            pltpu.VMEM((2, TOPK), jnp.float32),
            pltpu.VMEM((2, TOPK), jnp.int32),
            pltpu.SemaphoreType.DMA,
            pltpu.SemaphoreType.DMA,
            pltpu.SemaphoreType.DMA,
            pltpu.SemaphoreType.DMA,
            pltpu.SemaphoreType.DMA,
            pltpu.SemaphoreType.DMA,
            pltpu.SemaphoreType.DMA,
            pltpu.SemaphoreType.DMA,
        ],
        compiler_params=pltpu.CompilerParams(needs_layout_passes=False),
    )
    def body(lat, vals, idx, dense, rb0, rb1, rb2, rb3, zbuf0, zbuf1,
             lmaxA, lmaxB, svb0, sib0, svb1, sib1,
             sem0, sem1, sem2, sem3, semD0, semD1, semS0, semS1):
        iota = lax.iota(jnp.int32, 16)
        zero16 = jnp.zeros((16,), jnp.float32)
        zero16i = jnp.zeros((16,), jnp.int32)
        neg16 = jnp.full((16,), NEGINF, jnp.float32)
        wid = lax.axis_index("s") * 2 + lax.axis_index("c")
        base = wid * rpw

        def zero_body(i, carry):
            zbuf0[pl.ds(i * 16, 16)] = zero16
            zbuf1[pl.ds(i * 16, 16)] = zero16
            return carry

        lax.fori_loop(0, DS // 16, zero_body, 0)
        for jj in range(TOPK // 16):
            sib0[0, pl.ds(jj * 16, 16)] = zero16i
            sib0[1, pl.ds(jj * 16, 16)] = zero16i
            sib1[0, pl.ds(jj * 16, 16)] = zero16i
            sib1[1, pl.ds(jj * 16, 16)] = zero16i

        def pop_one(rbuf, lmax, sm, vv, ii, qi, qr):
            m = _rmax(sm)
            su = _scal(plsc.all_reduce_ffs(sm == m))
            lm = lmax[pl.ds(su * 16, 16)]
            lane = _scal(plsc.all_reduce_ffs(lm == m))
            cbase = su * 1024 + lane
            d = [plsc.load_gather(rbuf, [cbase + g * 256 + iota * 16])
                 for g in range(4)]
            jw = jnp.int32(1 << 20)
            for g in range(4):
                ff = _scal(plsc.all_reduce_ffs(d[g] == m))
                jw = jnp.minimum(jw, g * 16 + ff + ((ff & 16) << 8))
            gw = jw // 16
            tw = jw % 16
            flat = cbase + jw * 16
            for jj in range(4):
                sel = jnp.logical_and(qi == jj, iota == qr)
                vv[jj] = jnp.where(sel, m, vv[jj])
                ii[jj] = jnp.where(sel, flat, ii[jj])
            plsc.store_scatter(rbuf, [flat + iota * 0], neg16,
                               mask=iota == 0)
            d2 = [jnp.where(jnp.logical_and(gw == g, iota == tw),
                            NEGINF, d[g]) for g in range(4)]
            cm = _rmax(jnp.maximum(jnp.maximum(d2[0], d2[1]),
                                   jnp.maximum(d2[2], d2[3])))
            lm2 = jnp.where(iota == lane, cm, lm)
            lmax[pl.ds(su * 16, 16)] = lm2
            sm = jnp.where(iota == su, _rmax(lm2), sm)
            return sm, vv, ii

        def do_pair(rbA, rbB, rowA, rowB, sv, si, semS, sip,
                    dense_guard, stage_guard):
            def p1(s, carry):
                smA, smB = carry
                off = s * 1024
                aA = [rbA[pl.ds(off + j * 16, 16)] for j in range(4)]
                aB = [rbB[pl.ds(off + j * 16, 16)] for j in range(4)]
                for j in range(4, 64):
                    aA[j % 4] = jnp.maximum(aA[j % 4],
                                            rbA[pl.ds(off + j * 16, 16)])
                    aB[j % 4] = jnp.maximum(aB[j % 4],
                                            rbB[pl.ds(off + j * 16, 16)])
                mvA = jnp.maximum(jnp.maximum(aA[0], aA[1]),
                                  jnp.maximum(aA[2], aA[3]))
                mvB = jnp.maximum(jnp.maximum(aB[0], aB[1]),
                                  jnp.maximum(aB[2], aB[3]))
                lmaxA[pl.ds(s * 16, 16)] = mvA
                lmaxB[pl.ds(s * 16, 16)] = mvB
                smA = jnp.where(iota == s, _rmax(mvA), smA)
                smB = jnp.where(iota == s, _rmax(mvB), smB)
                return smA, smB

            smA, smB = lax.fori_loop(0, _NSUP, p1, (neg16, neg16))

            def pop(i, carry):
                smA = carry[0]
                vvA = list(carry[1:5])
                iiA = list(carry[5:9])
                smB = carry[9]
                vvB = list(carry[10:14])
                iiB = list(carry[14:18])
                qi = i // 16
                qr = i % 16
                smA, vvA, iiA = pop_one(rbA, lmaxA, smA, vvA, iiA, qi, qr)
                smB, vvB, iiB = pop_one(rbB, lmaxB, smB, vvB, iiB, qi, qr)
                return (smA,) + tuple(vvA) + tuple(iiA) + \
                       (smB,) + tuple(vvB) + tuple(iiB)

            init = (smA, zero16, zero16, zero16, zero16,
                    zero16i, zero16i, zero16i, zero16i,
                    smB, zero16, zero16, zero16, zero16,
                    zero16i, zero16i, zero16i, zero16i)
            res = lax.fori_loop(0, TOPK, pop, init)

            def wait_dense():
                pltpu.make_async_copy(zbuf0, dense.at[rowA], semD0).wait()
                pltpu.make_async_copy(zbuf1, dense.at[rowB], semD1).wait()

            if dense_guard is None:
                wait_dense()
            else:
                pl.when(dense_guard)(wait_dense)

            @pl.when(stage_guard)
            def _():
                pltpu.make_async_copy(sv, vals.at[pl.ds(rowA, 2)],
                                      semS).wait()
                pltpu.make_async_copy(si, idx.at[pl.ds(rowA, 2)],
                                      semS).wait()

            for jj in range(TOPK // 16):
                plsc.store_scatter(zbuf0, [sip[0, pl.ds(jj * 16, 16)]],
                                   zero16)
                plsc.store_scatter(zbuf1, [sip[1, pl.ds(jj * 16, 16)]],
                                   zero16)

            for (sl, zb, off) in ((0, zbuf0, 1), (1, zbuf1, 10)):
                for jj in range(TOPK // 16):
                    sv[sl, pl.ds(jj * 16, 16)] = res[off + jj]
                    si[sl, pl.ds(jj * 16, 16)] = res[off + 4 + jj]
                    plsc.store_scatter(zb, [res[off + 4 + jj]],
                                       res[off + jj])
            pltpu.async_copy(zbuf0, dense.at[rowA], semD0)
            pltpu.async_copy(zbuf1, dense.at[rowB], semD1)
            pltpu.async_copy(sv, vals.at[pl.ds(rowA, 2)], semS)
            pltpu.async_copy(si, idx.at[pl.ds(rowA, 2)], semS)

        pltpu.async_copy(lat.at[base], rb0, sem0)
        pltpu.async_copy(lat.at[base + 1], rb1, sem1)

        def outer(o, carry):
            r = o * 4
            pltpu.async_copy(lat.at[base + r + 2], rb2, sem2)
            pltpu.async_copy(lat.at[base + r + 3], rb3, sem3)
            pltpu.make_async_copy(lat.at[base + r], rb0, sem0).wait()
            pltpu.make_async_copy(lat.at[base + r + 1], rb1, sem1).wait()
            do_pair(rb0, rb1, base + r, base + r + 1,
                    svb0, sib0, semS0, sib1, o > 0, o > 0)

            @pl.when(o < rpw // 4 - 1)
            def _():
                pltpu.async_copy(lat.at[base + r + 4], rb0, sem0)
                pltpu.async_copy(lat.at[base + r + 5], rb1, sem1)

            pltpu.make_async_copy(lat.at[base + r + 2], rb2, sem2).wait()
            pltpu.make_async_copy(lat.at[base + r + 3], rb3, sem3).wait()
            do_pair(rb2, rb3, base + r + 2, base + r + 3,
                    svb1, sib1, semS1, sib0, None, o > 0)
            return carry

        lax.fori_loop(0, rpw // 4, outer, 0)

        pltpu.make_async_copy(zbuf0, dense.at[base], semD0).wait()
        pltpu.make_async_copy(zbuf1, dense.at[base], semD1).wait()
        pltpu.make_async_copy(svb0, vals.at[pl.ds(base, 2)], semS0).wait()
        pltpu.make_async_copy(sib0, idx.at[pl.ds(base, 2)], semS0).wait()
        pltpu.make_async_copy(svb1, vals.at[pl.ds(base, 2)], semS1).wait()
        pltpu.make_async_copy(sib1, idx.at[pl.ds(base, 2)], semS1).wait()

    return body(latents)


_NSLICE = 8


def kernel(x, W_enc, b_enc, W_dec, b_pre, k):
    del k
    rs = BATCH // _NSLICE
    Wd = W_dec.astype(jnp.bfloat16)
    vals_l, idx_l, xh_l = [], [], []
    dense_full = None
    for s in range(_NSLICE):
        lat = _encode(x[s * rs:(s + 1) * rs], W_enc, b_enc, b_pre, rs)
        v, i, d = _sc_topk(lat, rs)
        xh, dense_full = _decode_slice(d, Wd, b_pre, dense_full, s, rs)
        vals_l.append(v)
        idx_l.append(i)
        xh_l.append(xh)
    return (jnp.concatenate(xh_l), dense_full,
            jnp.concatenate(idx_l), jnp.concatenate(vals_l))
